# baseline (speedup 1.0000x reference)
import functools

import jax
import jax.numpy as jnp
from jax import lax
from jax.experimental import pallas as pl
from jax.experimental.pallas import tpu as pltpu
from jax.experimental.pallas import tpu_sc as plsc

N = 50000
NMAP = 128
NSCALES = 6
E = 30000
EL = 10000
NLAYERS = 4
NSLOT = 14
EPS = 1e-5

BLK = 8448
NBLK = 6
NPAD = BLK * NBLK
TRASH = BLK
T = 128
ETOT = 2 * NSCALES * E + 2 * EL
EPADN = ((ETOT + T - 1) // T + NBLK) * T
ROWS_PER_SUB = BLK // 16
ZCH = 48
BR = 1000
GRID = N // BR


def _tcin_body(x_ref, w8_ref, b8_ref, wi2_ref, ws2_ref, gb_ref, wm1_ref,
               wm2_ref, gbm_ref, o_ref):
    x = x_ref[...]
    a = jax.nn.relu(
        lax.dot_general(x, w8_ref[...], (((1,), (0,)), ((), ())),
                        preferred_element_type=jnp.float32) + b8_ref[...])
    h_in = lax.dot_general(a[:, :NMAP], wi2_ref[...], (((1,), (0,)), ((), ())),
                           preferred_element_type=jnp.float32)
    h_seg = lax.dot_general(a[:, NMAP:], ws2_ref[...], (((1,), (0,)), ((), ())),
                            preferred_element_type=jnp.float32)

    def gn(t, g, b):
        m = jnp.mean(t, axis=1, keepdims=True)
        v = jnp.mean((t - m) ** 2, axis=1, keepdims=True)
        return (t - m) * lax.rsqrt(v + EPS) * g + b

    gb = gb_ref[...]
    h_in = gn(h_in, gb[0:1, :], gb[1:2, :])
    h_seg = gn(h_seg, gb[2:3, :], gb[3:4, :])
    f = jax.nn.relu(h_in + h_seg)
    t = (lax.dot_general(f, wm1_ref[...], (((1,), (0,)), ((), ())),
                         preferred_element_type=jnp.float32)
         + lax.dot_general(x, wm2_ref[...], (((1,), (0,)), ((), ())),
                           preferred_element_type=jnp.float32))
    gbm = gbm_ref[...]
    o_ref[...] = jax.nn.relu(gn(t, gbm[0:1, :], gbm[1:2, :]))


def _tc_input(x, w8, b8, wi2, ws2, gb, wm1, wm2, gbm):
    return pl.pallas_call(
        _tcin_body,
        grid=(GRID,),
        in_specs=[
            pl.BlockSpec((BR, 8), lambda i: (i, 0)),
            pl.BlockSpec((8, 2 * NMAP), lambda i: (0, 0)),
            pl.BlockSpec((1, 2 * NMAP), lambda i: (0, 0)),
            pl.BlockSpec((NMAP, NMAP), lambda i: (0, 0)),
            pl.BlockSpec((NMAP, NMAP), lambda i: (0, 0)),
            pl.BlockSpec((4, NMAP), lambda i: (0, 0)),
            pl.BlockSpec((NMAP, NMAP), lambda i: (0, 0)),
            pl.BlockSpec((8, NMAP), lambda i: (0, 0)),
            pl.BlockSpec((2, NMAP), lambda i: (0, 0)),
        ],
        out_specs=pl.BlockSpec((BR, NMAP), lambda i: (i, 0)),
        out_shape=jax.ShapeDtypeStruct((N, NMAP), jnp.float32),
    )(x, w8, b8, wi2, ws2, gb, wm1, wm2, gbm)


def _tcmm_body(f_ref, we_ref, ze_ref):
    f = f_ref[...]
    for s in range(NSLOT):
        ze_ref[s] = lax.dot_general(f, we_ref[s], (((1,), (0,)), ((), ())),
                                    preferred_element_type=jnp.float32)


def _tc_matmul(feat, we):
    return pl.pallas_call(
        _tcmm_body,
        grid=(GRID,),
        in_specs=[
            pl.BlockSpec((BR, NMAP), lambda i: (i, 0)),
            pl.BlockSpec((NSLOT, NMAP, NMAP), lambda i: (0, 0, 0)),
        ],
        out_specs=pl.BlockSpec((NSLOT, BR, NMAP), lambda i: (0, i, 0)),
        out_shape=jax.ShapeDtypeStruct((NSLOT, N, NMAP), jnp.float32),
    )(feat, we)


def _sc_body(ze, sdgh, tloh, thih, out,
             sdg0, sdg1, liv0, liv1, rows0, rows1, zbuf, tlov, thiv, spmem,
             sem0, sem1):
    c = lax.axis_index("c")
    w = lax.axis_index("s")
    pltpu.sync_copy(tloh, tlov)
    pltpu.sync_copy(thih, thiv)
    zero16 = jnp.zeros((16,), jnp.float32)
    for r in range(ZCH):
        for cc in range(8):
            zbuf[r, pl.ds(cc * 16, 16)] = zero16

    tlo_all = tlov[...]
    thi_all = thiv[...]
    sdgs = (sdg0, sdg1)
    livs = (liv0, liv1)
    rowss = (rows0, rows1)
    sems = (sem0, sem1)

    def _fetch(t, slot):
        pltpu.sync_copy(sdgh.at[pl.ds(t * 2 * T, 2 * T)], sdgs[slot])
        pltpu.async_copy(ze.at[sdgs[slot].at[pl.ds(T, T)]], rowss[slot],
                         sems[slot])

    def _drain(base, slot):
        pltpu.make_async_copy(ze.at[sdgs[slot].at[pl.ds(T, T)]], rowss[slot],
                              sems[slot]).wait()
        for i in range(T // 16):
            d = sdgs[slot][pl.ds(i * 16, 16)]
            loc = d - base
            okm = (d >= base) & (d < base + BLK)
            livs[slot][pl.ds(i * 16, 16)] = jnp.where(okm, loc, TRASH)
        pltpu.sync_copy(rowss[slot], spmem.at[livs[slot]], add=True)

    for b in range(NBLK):
        @pl.when(c == b // (NBLK // 2))
        def _(b=b):
            base = b * BLK
            for kk in range(ROWS_PER_SUB // ZCH):
                pltpu.sync_copy(
                    zbuf, spmem.at[pl.ds(w * ROWS_PER_SUB + kk * ZCH, ZCH), :])
            plsc.subcore_barrier()

            tlo = tlo_all[b]
            thi = thi_all[b]
            nt = thi - tlo
            niter = jnp.maximum(nt - w + 15, 0) // 16

            @pl.when(niter > 0)
            def _():
                _fetch(tlo + w, 0)

                def pair_body(k, carry):
                    j0 = 2 * k
                    j1 = j0 + 1

                    @pl.when(j1 < niter)
                    def _():
                        _fetch(tlo + w + j1 * 16, 1)
                    _drain(base, 0)

                    @pl.when(j1 < niter)
                    def _():
                        @pl.when(j1 + 1 < niter)
                        def _():
                            _fetch(tlo + w + (j1 + 1) * 16, 0)
                        _drain(base, 1)
                    return carry

                lax.fori_loop(0, (niter + 1) // 2, pair_body, 0)
            plsc.subcore_barrier()
            pltpu.sync_copy(
                spmem.at[pl.ds(w * ROWS_PER_SUB, ROWS_PER_SUB), :],
                out.at[pl.ds(base + w * ROWS_PER_SUB, ROWS_PER_SUB), :])
            plsc.subcore_barrier()


@functools.cache
def _sc_scatter_fn():
    return pl.kernel(
        _sc_body,
        out_type=jax.ShapeDtypeStruct((NPAD, NMAP), jnp.float32),
        mesh=plsc.VectorSubcoreMesh(core_axis_name="c", subcore_axis_name="s"),
        scratch_types=[
            pltpu.VMEM((2 * T,), jnp.int32),
            pltpu.VMEM((2 * T,), jnp.int32),
            pltpu.VMEM((T,), jnp.int32),
            pltpu.VMEM((T,), jnp.int32),
            pltpu.VMEM((T, NMAP), jnp.float32),
            pltpu.VMEM((T, NMAP), jnp.float32),
            pltpu.VMEM((ZCH, NMAP), jnp.float32),
            pltpu.VMEM((16,), jnp.int32),
            pltpu.VMEM((16,), jnp.int32),
            pltpu.VMEM_SHARED((BLK + 8, NMAP), jnp.float32),
            pltpu.SemaphoreType.DMA,
            pltpu.SemaphoreType.DMA,
        ],
    )


def _sc_scatter(ze, sdg, tlo, thi):
    return _sc_scatter_fn()(ze, sdg, tlo, thi)


def _tcpost_body(s_ref, res_ref, wc_ref, w2_ref, gb_ref, o_ref):
    res = res_ref[...]
    t = s_ref[...] + lax.dot_general(res, wc_ref[...], (((1,), (0,)), ((), ())),
                                     preferred_element_type=jnp.float32)

    def gn(t, g, b):
        m = jnp.mean(t, axis=1, keepdims=True)
        v = jnp.mean((t - m) ** 2, axis=1, keepdims=True)
        return (t - m) * lax.rsqrt(v + EPS) * g + b

    gb = gb_ref[...]
    a = jax.nn.relu(gn(t, gb[0:1, :], gb[1:2, :]))
    h = lax.dot_general(a, w2_ref[...], (((1,), (0,)), ((), ())),
                        preferred_element_type=jnp.float32)
    h = gn(h, gb[2:3, :], gb[3:4, :])
    o_ref[...] = jax.nn.relu(h + res)


def _tc_post(s, res, wc, w2, gb):
    return pl.pallas_call(
        _tcpost_body,
        grid=(GRID,),
        in_specs=[
            pl.BlockSpec((BR, NMAP), lambda i: (i, 0)),
            pl.BlockSpec((BR, NMAP), lambda i: (i, 0)),
            pl.BlockSpec((NMAP, NMAP), lambda i: (0, 0)),
            pl.BlockSpec((NMAP, NMAP), lambda i: (0, 0)),
            pl.BlockSpec((4, NMAP), lambda i: (0, 0)),
        ],
        out_specs=pl.BlockSpec((BR, NMAP), lambda i: (i, 0)),
        out_shape=jax.ShapeDtypeStruct((N, NMAP), jnp.float32),
    )(s, res, wc, w2, gb)


def kernel(control, pre, right, suc, turn, intersect, ctrs, feats, left,
           Wi1, bi1, Wi2, gi, bi, Ws1, bs1, Ws2, gs, bs, Wm, gm, bm,
           ctr_W, pre_W, suc_W, left_W, right_W, norm_g, norm_b,
           ctr2_W, ctr2_g, ctr2_b):
    f32 = jnp.float32
    x = jnp.concatenate([ctrs, feats, turn, control[:, None],
                         intersect[:, None]], axis=1).astype(f32)
    w8 = jnp.zeros((8, 2 * NMAP), f32)
    w8 = w8.at[0:2, :NMAP].set(Wi1).at[2:4, NMAP:].set(Ws1)
    b8 = jnp.concatenate([bi1, bs1])[None, :]
    gb_in = jnp.stack([gi, bi, gs, bs])
    wm1 = Wm[:NMAP]
    wm2 = jnp.zeros((8, NMAP), f32).at[4:8, :].set(Wm[NMAP:NMAP + 4])
    gbm = jnp.stack([gm, bm])
    we = jnp.concatenate([
        pre_W, suc_W, left_W[:, None], right_W[:, None]], axis=1)
    gb_post = jnp.stack([norm_g, norm_b, ctr2_g, ctr2_b], axis=1)

    dsts = jnp.concatenate([pre[:, 0].reshape(-1), suc[:, 0].reshape(-1),
                            left[0], right[0]])
    srcs = jnp.concatenate([pre[:, 1].reshape(-1), suc[:, 1].reshape(-1),
                            left[1], right[1]])
    slots = jnp.concatenate([
        jnp.repeat(jnp.arange(NSCALES, dtype=jnp.int32), E),
        NSCALES + jnp.repeat(jnp.arange(NSCALES, dtype=jnp.int32), E),
        jnp.full((EL,), 2 * NSCALES, jnp.int32),
        jnp.full((EL,), 2 * NSCALES + 1, jnp.int32)])
    g = slots * N + srcs
    bid = dsts // BLK
    masks = [bid == b for b in range(NBLK)]
    csums = [jnp.cumsum(m.astype(jnp.int32)) for m in masks]
    counts = jnp.stack([cs[-1] for cs in csums])
    offs = jnp.concatenate([jnp.zeros((1,), jnp.int32),
                            jnp.cumsum((counts + T - 1) // T * T)])
    pos = jnp.zeros((ETOT,), jnp.int32)
    for b in range(NBLK):
        pos = jnp.where(masks[b], offs[b] + csums[b] - 1, pos)
    init = jnp.concatenate([
        jnp.full((EPADN, 1), NPAD - 1, jnp.int32),
        jnp.zeros((EPADN, 1), jnp.int32)], axis=1)
    rows = jnp.stack([dsts, g], axis=1)
    sorted_rows = init.at[pos].set(rows, unique_indices=True, mode="drop")
    sd = sorted_rows[:, 0]
    sg = sorted_rows[:, 1]
    sdg = jnp.stack([sd.reshape(-1, T), sg.reshape(-1, T)], axis=1).reshape(-1)
    tlo = jnp.zeros((16,), jnp.int32).at[:NBLK].set(offs[:NBLK] // T)
    thi = jnp.zeros((16,), jnp.int32).at[:NBLK].set(
        (offs[:NBLK] + counts + T - 1) // T)

    feat = _tc_input(x, w8, b8, Wi2, Ws2, gb_in, wm1, wm2, gbm)
    for i in range(NLAYERS):
        zeall = _tc_matmul(feat, we[i])
        s = _sc_scatter(zeall.reshape(NSLOT * N, NMAP), sdg, tlo, thi)
        feat = _tc_post(s, feat, ctr_W[i], ctr2_W[i], gb_post[i])
    return (feat, ctrs)


# speedup vs baseline: 2.8602x; 2.8602x over previous
import functools

import jax
import jax.numpy as jnp
from jax import lax
from jax.experimental import pallas as pl
from jax.experimental.pallas import tpu as pltpu
from jax.experimental.pallas import tpu_sc as plsc

N = 50000
NMAP = 128
NSCALES = 6
E = 30000
EL = 10000
NLAYERS = 4
NSLOT = 14
EPS = 1e-5

BLK = 8448
NBLK = 6
NPAD = BLK * NBLK
TRASH = BLK
T = 128
ETOT = 2 * NSCALES * E + 2 * EL
EPADN = ((ETOT + T - 1) // T + NBLK) * T
ROWS_PER_SUB = BLK // 16
ZCH = 48
BR = 1000
GRID = N // BR


def _tcin_body(x_ref, w8_ref, b8_ref, wi2_ref, ws2_ref, gb_ref, wm1_ref,
               wm2_ref, gbm_ref, o_ref):
    x = x_ref[...]
    a = jax.nn.relu(
        lax.dot_general(x, w8_ref[...], (((1,), (0,)), ((), ())),
                        preferred_element_type=jnp.float32) + b8_ref[...])
    h_in = lax.dot_general(a[:, :NMAP], wi2_ref[...], (((1,), (0,)), ((), ())),
                           preferred_element_type=jnp.float32)
    h_seg = lax.dot_general(a[:, NMAP:], ws2_ref[...], (((1,), (0,)), ((), ())),
                            preferred_element_type=jnp.float32)

    def gn(t, g, b):
        m = jnp.mean(t, axis=1, keepdims=True)
        v = jnp.mean((t - m) ** 2, axis=1, keepdims=True)
        return (t - m) * lax.rsqrt(v + EPS) * g + b

    gb = gb_ref[...]
    h_in = gn(h_in, gb[0:1, :], gb[1:2, :])
    h_seg = gn(h_seg, gb[2:3, :], gb[3:4, :])
    f = jax.nn.relu(h_in + h_seg)
    t = (lax.dot_general(f, wm1_ref[...], (((1,), (0,)), ((), ())),
                         preferred_element_type=jnp.float32)
         + lax.dot_general(x, wm2_ref[...], (((1,), (0,)), ((), ())),
                           preferred_element_type=jnp.float32))
    gbm = gbm_ref[...]
    o_ref[...] = jax.nn.relu(gn(t, gbm[0:1, :], gbm[1:2, :]))


def _tc_input(x, w8, b8, wi2, ws2, gb, wm1, wm2, gbm):
    return pl.pallas_call(
        _tcin_body,
        grid=(GRID,),
        in_specs=[
            pl.BlockSpec((BR, 8), lambda i: (i, 0)),
            pl.BlockSpec((8, 2 * NMAP), lambda i: (0, 0)),
            pl.BlockSpec((1, 2 * NMAP), lambda i: (0, 0)),
            pl.BlockSpec((NMAP, NMAP), lambda i: (0, 0)),
            pl.BlockSpec((NMAP, NMAP), lambda i: (0, 0)),
            pl.BlockSpec((4, NMAP), lambda i: (0, 0)),
            pl.BlockSpec((NMAP, NMAP), lambda i: (0, 0)),
            pl.BlockSpec((8, NMAP), lambda i: (0, 0)),
            pl.BlockSpec((2, NMAP), lambda i: (0, 0)),
        ],
        out_specs=pl.BlockSpec((BR, NMAP), lambda i: (i, 0)),
        out_shape=jax.ShapeDtypeStruct((N, NMAP), jnp.float32),
    )(x, w8, b8, wi2, ws2, gb, wm1, wm2, gbm)


def _tcmm_body(f_ref, we_ref, ze_ref):
    f = f_ref[...]
    for s in range(NSLOT):
        ze_ref[s] = lax.dot_general(f, we_ref[s], (((1,), (0,)), ((), ())),
                                    preferred_element_type=jnp.float32)


def _tc_matmul(feat, we):
    return pl.pallas_call(
        _tcmm_body,
        grid=(GRID,),
        in_specs=[
            pl.BlockSpec((BR, NMAP), lambda i: (i, 0)),
            pl.BlockSpec((NSLOT, NMAP, NMAP), lambda i: (0, 0, 0)),
        ],
        out_specs=pl.BlockSpec((NSLOT, BR, NMAP), lambda i: (0, i, 0)),
        out_shape=jax.ShapeDtypeStruct((NSLOT, N, NMAP), jnp.float32),
    )(feat, we)


def _sc_body(ze, sdgh, tloh, thih, out,
             sdg0, sdg1, liv0, liv1, rows0, rows1, zbuf, tlov, thiv, spmem,
             sem0, sem1):
    c = lax.axis_index("c")
    w = lax.axis_index("s")
    pltpu.sync_copy(tloh, tlov)
    pltpu.sync_copy(thih, thiv)
    zero16 = jnp.zeros((16,), jnp.float32)
    for r in range(ZCH):
        for cc in range(8):
            zbuf[r, pl.ds(cc * 16, 16)] = zero16

    tlo_all = tlov[...]
    thi_all = thiv[...]
    sdgs = (sdg0, sdg1)
    livs = (liv0, liv1)
    rowss = (rows0, rows1)
    sems = (sem0, sem1)

    def _fetch(t, slot):
        pltpu.sync_copy(sdgh.at[pl.ds(t * 2 * T, 2 * T)], sdgs[slot])
        pltpu.async_copy(ze.at[sdgs[slot].at[pl.ds(T, T)]], rowss[slot],
                         sems[slot])

    def _drain(base, slot):
        pltpu.make_async_copy(ze.at[sdgs[slot].at[pl.ds(T, T)]], rowss[slot],
                              sems[slot]).wait()
        for i in range(T // 16):
            d = sdgs[slot][pl.ds(i * 16, 16)]
            loc = d - base
            okm = (d >= base) & (d < base + BLK)
            livs[slot][pl.ds(i * 16, 16)] = jnp.where(okm, loc, TRASH)
        pltpu.sync_copy(rowss[slot], spmem.at[livs[slot]], add=True)

    for b in range(NBLK):
        @pl.when(c == b // (NBLK // 2))
        def _(b=b):
            base = b * BLK
            for kk in range(ROWS_PER_SUB // ZCH):
                pltpu.sync_copy(
                    zbuf, spmem.at[pl.ds(w * ROWS_PER_SUB + kk * ZCH, ZCH), :])
            plsc.subcore_barrier()

            tlo = tlo_all[b]
            thi = thi_all[b]
            nt = thi - tlo
            niter = jnp.maximum(nt - w + 15, 0) // 16

            @pl.when(niter > 0)
            def _():
                _fetch(tlo + w, 0)

                def pair_body(k, carry):
                    j0 = 2 * k
                    j1 = j0 + 1

                    @pl.when(j1 < niter)
                    def _():
                        _fetch(tlo + w + j1 * 16, 1)
                    _drain(base, 0)

                    @pl.when(j1 < niter)
                    def _():
                        @pl.when(j1 + 1 < niter)
                        def _():
                            _fetch(tlo + w + (j1 + 1) * 16, 0)
                        _drain(base, 1)
                    return carry

                lax.fori_loop(0, (niter + 1) // 2, pair_body, 0)
            plsc.subcore_barrier()
            pltpu.sync_copy(
                spmem.at[pl.ds(w * ROWS_PER_SUB, ROWS_PER_SUB), :],
                out.at[pl.ds(base + w * ROWS_PER_SUB, ROWS_PER_SUB), :])
            plsc.subcore_barrier()


@functools.cache
def _sc_scatter_fn():
    return pl.kernel(
        _sc_body,
        out_type=jax.ShapeDtypeStruct((NPAD, NMAP), jnp.float32),
        mesh=plsc.VectorSubcoreMesh(core_axis_name="c", subcore_axis_name="s"),
        scratch_types=[
            pltpu.VMEM((2 * T,), jnp.int32),
            pltpu.VMEM((2 * T,), jnp.int32),
            pltpu.VMEM((T,), jnp.int32),
            pltpu.VMEM((T,), jnp.int32),
            pltpu.VMEM((T, NMAP), jnp.float32),
            pltpu.VMEM((T, NMAP), jnp.float32),
            pltpu.VMEM((ZCH, NMAP), jnp.float32),
            pltpu.VMEM((16,), jnp.int32),
            pltpu.VMEM((16,), jnp.int32),
            pltpu.VMEM_SHARED((BLK + 8, NMAP), jnp.float32),
            pltpu.SemaphoreType.DMA,
            pltpu.SemaphoreType.DMA,
        ],
    )


def _sc_scatter(ze, sdg, tlo, thi):
    return _sc_scatter_fn()(ze, sdg, tlo, thi)


def _tcpost_body(s_ref, res_ref, wc_ref, w2_ref, gb_ref, o_ref):
    res = res_ref[...]
    t = s_ref[...] + lax.dot_general(res, wc_ref[...], (((1,), (0,)), ((), ())),
                                     preferred_element_type=jnp.float32)

    def gn(t, g, b):
        m = jnp.mean(t, axis=1, keepdims=True)
        v = jnp.mean((t - m) ** 2, axis=1, keepdims=True)
        return (t - m) * lax.rsqrt(v + EPS) * g + b

    gb = gb_ref[...]
    a = jax.nn.relu(gn(t, gb[0:1, :], gb[1:2, :]))
    h = lax.dot_general(a, w2_ref[...], (((1,), (0,)), ((), ())),
                        preferred_element_type=jnp.float32)
    h = gn(h, gb[2:3, :], gb[3:4, :])
    o_ref[...] = jax.nn.relu(h + res)


def _tc_post(s, res, wc, w2, gb):
    return pl.pallas_call(
        _tcpost_body,
        grid=(GRID,),
        in_specs=[
            pl.BlockSpec((BR, NMAP), lambda i: (i, 0)),
            pl.BlockSpec((BR, NMAP), lambda i: (i, 0)),
            pl.BlockSpec((NMAP, NMAP), lambda i: (0, 0)),
            pl.BlockSpec((NMAP, NMAP), lambda i: (0, 0)),
            pl.BlockSpec((4, NMAP), lambda i: (0, 0)),
        ],
        out_specs=pl.BlockSpec((BR, NMAP), lambda i: (i, 0)),
        out_shape=jax.ShapeDtypeStruct((N, NMAP), jnp.float32),
    )(s, res, wc, w2, gb)


def kernel(control, pre, right, suc, turn, intersect, ctrs, feats, left,
           Wi1, bi1, Wi2, gi, bi, Ws1, bs1, Ws2, gs, bs, Wm, gm, bm,
           ctr_W, pre_W, suc_W, left_W, right_W, norm_g, norm_b,
           ctr2_W, ctr2_g, ctr2_b):
    f32 = jnp.float32
    x = jnp.concatenate([ctrs, feats, turn, control[:, None],
                         intersect[:, None]], axis=1).astype(f32)
    w8 = jnp.zeros((8, 2 * NMAP), f32)
    w8 = w8.at[0:2, :NMAP].set(Wi1).at[2:4, NMAP:].set(Ws1)
    b8 = jnp.concatenate([bi1, bs1])[None, :]
    gb_in = jnp.stack([gi, bi, gs, bs])
    wm1 = Wm[:NMAP]
    wm2 = jnp.zeros((8, NMAP), f32).at[4:8, :].set(Wm[NMAP:NMAP + 4])
    gbm = jnp.stack([gm, bm])
    we = jnp.concatenate([
        pre_W, suc_W, left_W[:, None], right_W[:, None]], axis=1)
    gb_post = jnp.stack([norm_g, norm_b, ctr2_g, ctr2_b], axis=1)

    dsts = jnp.concatenate([pre[:, 0].reshape(-1), suc[:, 0].reshape(-1),
                            left[0], right[0]])
    srcs = jnp.concatenate([pre[:, 1].reshape(-1), suc[:, 1].reshape(-1),
                            left[1], right[1]])
    slots = jnp.concatenate([
        jnp.repeat(jnp.arange(NSCALES, dtype=jnp.int32), E),
        NSCALES + jnp.repeat(jnp.arange(NSCALES, dtype=jnp.int32), E),
        jnp.full((EL,), 2 * NSCALES, jnp.int32),
        jnp.full((EL,), 2 * NSCALES + 1, jnp.int32)])
    g = slots * N + srcs
    sd, sg = lax.sort_key_val(dsts, g)
    sd = jnp.concatenate([sd, jnp.full((EPADN - ETOT,), NPAD - 1, jnp.int32)])
    sg = jnp.concatenate([sg, jnp.zeros((EPADN - ETOT,), jnp.int32)])
    sdg = jnp.stack([sd.reshape(-1, T), sg.reshape(-1, T)], axis=1).reshape(-1)
    bounds = jnp.searchsorted(
        sd, jnp.arange(NBLK + 1, dtype=jnp.int32) * BLK).astype(jnp.int32)
    tlo = jnp.zeros((16,), jnp.int32).at[:NBLK].set(bounds[:NBLK] // T)
    thi = jnp.zeros((16,), jnp.int32).at[:NBLK].set(
        (bounds[1:] + T - 1) // T)

    feat = _tc_input(x, w8, b8, Wi2, Ws2, gb_in, wm1, wm2, gbm)
    for i in range(NLAYERS):
        zeall = _tc_matmul(feat, we[i])
        s = _sc_scatter(zeall.reshape(NSLOT * N, NMAP), sdg, tlo, thi)
        feat = _tc_post(s, feat, ctr_W[i], ctr2_W[i], gb_post[i])
    return (feat, ctrs)


# speedup vs baseline: 3.0774x; 1.0760x over previous
import functools

import jax
import jax.numpy as jnp
from jax import lax
from jax.experimental import pallas as pl
from jax.experimental.pallas import tpu as pltpu
from jax.experimental.pallas import tpu_sc as plsc

N = 50000
NMAP = 128
NSCALES = 6
E = 30000
EL = 10000
NLAYERS = 4
NSLOT = 14
EPS = 1e-5

BLK = 8448
NBLK = 6
NPAD = BLK * NBLK
TRASH = BLK
T = 128
ETOT = 2 * NSCALES * E + 2 * EL
EPADN = ((ETOT + T - 1) // T + NBLK) * T
ROWS_PER_SUB = BLK // 16
ZCH = 48
BR = 1000
GRID = N // BR


def _tcin_body(x_ref, w8_ref, b8_ref, wi2_ref, ws2_ref, gb_ref, wm1_ref,
               wm2_ref, gbm_ref, we_ref, o_ref, ze_ref):
    x = x_ref[...]
    a = jax.nn.relu(
        lax.dot_general(x, w8_ref[...], (((1,), (0,)), ((), ())),
                        preferred_element_type=jnp.float32) + b8_ref[...])
    h_in = lax.dot_general(a[:, :NMAP], wi2_ref[...], (((1,), (0,)), ((), ())),
                           preferred_element_type=jnp.float32)
    h_seg = lax.dot_general(a[:, NMAP:], ws2_ref[...], (((1,), (0,)), ((), ())),
                            preferred_element_type=jnp.float32)

    def gn(t, g, b):
        m = jnp.mean(t, axis=1, keepdims=True)
        v = jnp.mean((t - m) ** 2, axis=1, keepdims=True)
        return (t - m) * lax.rsqrt(v + EPS) * g + b

    gb = gb_ref[...]
    h_in = gn(h_in, gb[0:1, :], gb[1:2, :])
    h_seg = gn(h_seg, gb[2:3, :], gb[3:4, :])
    f = jax.nn.relu(h_in + h_seg)
    t = (lax.dot_general(f, wm1_ref[...], (((1,), (0,)), ((), ())),
                         preferred_element_type=jnp.float32)
         + lax.dot_general(x, wm2_ref[...], (((1,), (0,)), ((), ())),
                           preferred_element_type=jnp.float32))
    gbm = gbm_ref[...]
    f = jax.nn.relu(gn(t, gbm[0:1, :], gbm[1:2, :]))
    o_ref[...] = f
    for sl in range(NSLOT):
        ze_ref[sl] = lax.dot_general(f, we_ref[sl], (((1,), (0,)), ((), ())),
                                     preferred_element_type=jnp.float32)


def _tc_input(x, w8, b8, wi2, ws2, gb, wm1, wm2, gbm, we):
    return pl.pallas_call(
        _tcin_body,
        grid=(GRID,),
        in_specs=[
            pl.BlockSpec((BR, 8), lambda i: (i, 0)),
            pl.BlockSpec((8, 2 * NMAP), lambda i: (0, 0)),
            pl.BlockSpec((1, 2 * NMAP), lambda i: (0, 0)),
            pl.BlockSpec((NMAP, NMAP), lambda i: (0, 0)),
            pl.BlockSpec((NMAP, NMAP), lambda i: (0, 0)),
            pl.BlockSpec((4, NMAP), lambda i: (0, 0)),
            pl.BlockSpec((NMAP, NMAP), lambda i: (0, 0)),
            pl.BlockSpec((8, NMAP), lambda i: (0, 0)),
            pl.BlockSpec((2, NMAP), lambda i: (0, 0)),
            pl.BlockSpec((NSLOT, NMAP, NMAP), lambda i: (0, 0, 0)),
        ],
        out_specs=[
            pl.BlockSpec((BR, NMAP), lambda i: (i, 0)),
            pl.BlockSpec((NSLOT, BR, NMAP), lambda i: (0, i, 0)),
        ],
        out_shape=[
            jax.ShapeDtypeStruct((N, NMAP), jnp.float32),
            jax.ShapeDtypeStruct((NSLOT, N, NMAP), jnp.float32),
        ],
    )(x, w8, b8, wi2, ws2, gb, wm1, wm2, gbm, we)


def _tcmm_body(f_ref, we_ref, ze_ref):
    f = f_ref[...]
    for s in range(NSLOT):
        ze_ref[s] = lax.dot_general(f, we_ref[s], (((1,), (0,)), ((), ())),
                                    preferred_element_type=jnp.float32)


def _tc_matmul(feat, we):
    return pl.pallas_call(
        _tcmm_body,
        grid=(GRID,),
        in_specs=[
            pl.BlockSpec((BR, NMAP), lambda i: (i, 0)),
            pl.BlockSpec((NSLOT, NMAP, NMAP), lambda i: (0, 0, 0)),
        ],
        out_specs=pl.BlockSpec((NSLOT, BR, NMAP), lambda i: (0, i, 0)),
        out_shape=jax.ShapeDtypeStruct((NSLOT, N, NMAP), jnp.float32),
    )(feat, we)


def _sc_body(ze, sdgh, tloh, thih, out,
             sdg0, sdg1, liv0, liv1, rows0, rows1, zbuf, tlov, thiv, spmem,
             sem0, sem1):
    c = lax.axis_index("c")
    w = lax.axis_index("s")
    pltpu.sync_copy(tloh, tlov)
    pltpu.sync_copy(thih, thiv)
    zero16 = jnp.zeros((16,), jnp.float32)
    for r in range(ZCH):
        for cc in range(8):
            zbuf[r, pl.ds(cc * 16, 16)] = zero16

    tlo_all = tlov[...]
    thi_all = thiv[...]
    sdgs = (sdg0, sdg1)
    livs = (liv0, liv1)
    rowss = (rows0, rows1)
    sems = (sem0, sem1)

    def _fetch(t, slot):
        pltpu.sync_copy(sdgh.at[pl.ds(t * 2 * T, 2 * T)], sdgs[slot])
        pltpu.async_copy(ze.at[sdgs[slot].at[pl.ds(T, T)]], rowss[slot],
                         sems[slot])

    def _drain(base, slot):
        pltpu.make_async_copy(ze.at[sdgs[slot].at[pl.ds(T, T)]], rowss[slot],
                              sems[slot]).wait()
        for i in range(T // 16):
            d = sdgs[slot][pl.ds(i * 16, 16)]
            loc = d - base
            okm = (d >= base) & (d < base + BLK)
            livs[slot][pl.ds(i * 16, 16)] = jnp.where(okm, loc, TRASH)
        pltpu.sync_copy(rowss[slot], spmem.at[livs[slot]], add=True)

    for b in range(NBLK):
        @pl.when(c == b // (NBLK // 2))
        def _(b=b):
            base = b * BLK
            for kk in range(ROWS_PER_SUB // ZCH):
                pltpu.sync_copy(
                    zbuf, spmem.at[pl.ds(w * ROWS_PER_SUB + kk * ZCH, ZCH), :])
            plsc.subcore_barrier()

            tlo = tlo_all[b]
            thi = thi_all[b]
            nt = thi - tlo
            niter = jnp.maximum(nt - w + 15, 0) // 16

            @pl.when(niter > 0)
            def _():
                _fetch(tlo + w, 0)

                def pair_body(k, carry):
                    j0 = 2 * k
                    j1 = j0 + 1

                    @pl.when(j1 < niter)
                    def _():
                        _fetch(tlo + w + j1 * 16, 1)
                    _drain(base, 0)

                    @pl.when(j1 < niter)
                    def _():
                        @pl.when(j1 + 1 < niter)
                        def _():
                            _fetch(tlo + w + (j1 + 1) * 16, 0)
                        _drain(base, 1)
                    return carry

                lax.fori_loop(0, (niter + 1) // 2, pair_body, 0)
            plsc.subcore_barrier()
            pltpu.sync_copy(
                spmem.at[pl.ds(w * ROWS_PER_SUB, ROWS_PER_SUB), :],
                out.at[pl.ds(base + w * ROWS_PER_SUB, ROWS_PER_SUB), :])
            plsc.subcore_barrier()


@functools.cache
def _sc_scatter_fn():
    return pl.kernel(
        _sc_body,
        out_type=jax.ShapeDtypeStruct((NPAD, NMAP), jnp.float32),
        mesh=plsc.VectorSubcoreMesh(core_axis_name="c", subcore_axis_name="s"),
        scratch_types=[
            pltpu.VMEM((2 * T,), jnp.int32),
            pltpu.VMEM((2 * T,), jnp.int32),
            pltpu.VMEM((T,), jnp.int32),
            pltpu.VMEM((T,), jnp.int32),
            pltpu.VMEM((T, NMAP), jnp.float32),
            pltpu.VMEM((T, NMAP), jnp.float32),
            pltpu.VMEM((ZCH, NMAP), jnp.float32),
            pltpu.VMEM((16,), jnp.int32),
            pltpu.VMEM((16,), jnp.int32),
            pltpu.VMEM_SHARED((BLK + 8, NMAP), jnp.float32),
            pltpu.SemaphoreType.DMA,
            pltpu.SemaphoreType.DMA,
        ],
    )


def _sc_scatter(ze, sdg, tlo, thi):
    return _sc_scatter_fn()(ze, sdg, tlo, thi)


def _tcpost_body(s_ref, res_ref, wc_ref, w2_ref, gb_ref, o_ref):
    res = res_ref[...]
    t = s_ref[...] + lax.dot_general(res, wc_ref[...], (((1,), (0,)), ((), ())),
                                     preferred_element_type=jnp.float32)

    def gn(t, g, b):
        m = jnp.mean(t, axis=1, keepdims=True)
        v = jnp.mean((t - m) ** 2, axis=1, keepdims=True)
        return (t - m) * lax.rsqrt(v + EPS) * g + b

    gb = gb_ref[...]
    a = jax.nn.relu(gn(t, gb[0:1, :], gb[1:2, :]))
    h = lax.dot_general(a, w2_ref[...], (((1,), (0,)), ((), ())),
                        preferred_element_type=jnp.float32)
    h = gn(h, gb[2:3, :], gb[3:4, :])
    o_ref[...] = jax.nn.relu(h + res)


def _tcpostmm_body(s_ref, res_ref, wc_ref, w2_ref, gb_ref, we_ref,
                   o_ref, ze_ref):
    res = res_ref[...]
    t = s_ref[...] + lax.dot_general(res, wc_ref[...], (((1,), (0,)), ((), ())),
                                     preferred_element_type=jnp.float32)

    def gn(t, g, b):
        m = jnp.mean(t, axis=1, keepdims=True)
        v = jnp.mean((t - m) ** 2, axis=1, keepdims=True)
        return (t - m) * lax.rsqrt(v + EPS) * g + b

    gb = gb_ref[...]
    a = jax.nn.relu(gn(t, gb[0:1, :], gb[1:2, :]))
    h = lax.dot_general(a, w2_ref[...], (((1,), (0,)), ((), ())),
                        preferred_element_type=jnp.float32)
    h = gn(h, gb[2:3, :], gb[3:4, :])
    f = jax.nn.relu(h + res)
    o_ref[...] = f
    for sl in range(NSLOT):
        ze_ref[sl] = lax.dot_general(f, we_ref[sl], (((1,), (0,)), ((), ())),
                                     preferred_element_type=jnp.float32)


def _tc_post_mm(s, res, wc, w2, gb, we):
    return pl.pallas_call(
        _tcpostmm_body,
        grid=(GRID,),
        in_specs=[
            pl.BlockSpec((BR, NMAP), lambda i: (i, 0)),
            pl.BlockSpec((BR, NMAP), lambda i: (i, 0)),
            pl.BlockSpec((NMAP, NMAP), lambda i: (0, 0)),
            pl.BlockSpec((NMAP, NMAP), lambda i: (0, 0)),
            pl.BlockSpec((4, NMAP), lambda i: (0, 0)),
            pl.BlockSpec((NSLOT, NMAP, NMAP), lambda i: (0, 0, 0)),
        ],
        out_specs=[
            pl.BlockSpec((BR, NMAP), lambda i: (i, 0)),
            pl.BlockSpec((NSLOT, BR, NMAP), lambda i: (0, i, 0)),
        ],
        out_shape=[
            jax.ShapeDtypeStruct((N, NMAP), jnp.float32),
            jax.ShapeDtypeStruct((NSLOT, N, NMAP), jnp.float32),
        ],
    )(s, res, wc, w2, gb, we)


def _tc_post(s, res, wc, w2, gb):
    return pl.pallas_call(
        _tcpost_body,
        grid=(GRID,),
        in_specs=[
            pl.BlockSpec((BR, NMAP), lambda i: (i, 0)),
            pl.BlockSpec((BR, NMAP), lambda i: (i, 0)),
            pl.BlockSpec((NMAP, NMAP), lambda i: (0, 0)),
            pl.BlockSpec((NMAP, NMAP), lambda i: (0, 0)),
            pl.BlockSpec((4, NMAP), lambda i: (0, 0)),
        ],
        out_specs=pl.BlockSpec((BR, NMAP), lambda i: (i, 0)),
        out_shape=jax.ShapeDtypeStruct((N, NMAP), jnp.float32),
    )(s, res, wc, w2, gb)


def kernel(control, pre, right, suc, turn, intersect, ctrs, feats, left,
           Wi1, bi1, Wi2, gi, bi, Ws1, bs1, Ws2, gs, bs, Wm, gm, bm,
           ctr_W, pre_W, suc_W, left_W, right_W, norm_g, norm_b,
           ctr2_W, ctr2_g, ctr2_b):
    f32 = jnp.float32
    x = jnp.concatenate([ctrs, feats, turn, control[:, None],
                         intersect[:, None]], axis=1).astype(f32)
    w8 = jnp.zeros((8, 2 * NMAP), f32)
    w8 = w8.at[0:2, :NMAP].set(Wi1).at[2:4, NMAP:].set(Ws1)
    b8 = jnp.concatenate([bi1, bs1])[None, :]
    gb_in = jnp.stack([gi, bi, gs, bs])
    wm1 = Wm[:NMAP]
    wm2 = jnp.zeros((8, NMAP), f32).at[4:8, :].set(Wm[NMAP:NMAP + 4])
    gbm = jnp.stack([gm, bm])
    we = jnp.concatenate([
        pre_W, suc_W, left_W[:, None], right_W[:, None]], axis=1)
    gb_post = jnp.stack([norm_g, norm_b, ctr2_g, ctr2_b], axis=1)

    dsts = jnp.concatenate([pre[:, 0].reshape(-1), suc[:, 0].reshape(-1),
                            left[0], right[0]])
    srcs = jnp.concatenate([pre[:, 1].reshape(-1), suc[:, 1].reshape(-1),
                            left[1], right[1]])
    slots = jnp.concatenate([
        jnp.repeat(jnp.arange(NSCALES, dtype=jnp.int32), E),
        NSCALES + jnp.repeat(jnp.arange(NSCALES, dtype=jnp.int32), E),
        jnp.full((EL,), 2 * NSCALES, jnp.int32),
        jnp.full((EL,), 2 * NSCALES + 1, jnp.int32)])
    g = slots * N + srcs
    sd, sg = lax.sort_key_val(dsts, g)
    sd = jnp.concatenate([sd, jnp.full((EPADN - ETOT,), NPAD - 1, jnp.int32)])
    sg = jnp.concatenate([sg, jnp.zeros((EPADN - ETOT,), jnp.int32)])
    sdg = jnp.stack([sd.reshape(-1, T), sg.reshape(-1, T)], axis=1).reshape(-1)
    bounds = jnp.searchsorted(
        sd, jnp.arange(NBLK + 1, dtype=jnp.int32) * BLK).astype(jnp.int32)
    tlo = jnp.zeros((16,), jnp.int32).at[:NBLK].set(bounds[:NBLK] // T)
    thi = jnp.zeros((16,), jnp.int32).at[:NBLK].set(
        (bounds[1:] + T - 1) // T)

    feat, ze = _tc_input(x, w8, b8, Wi2, Ws2, gb_in, wm1, wm2, gbm, we[0])
    for i in range(NLAYERS):
        sacc = _sc_scatter(ze.reshape(NSLOT * N, NMAP), sdg, tlo, thi)
        if i < NLAYERS - 1:
            feat, ze = _tc_post_mm(sacc, feat, ctr_W[i], ctr2_W[i],
                                   gb_post[i], we[i + 1])
        else:
            feat = _tc_post(sacc, feat, ctr_W[i], ctr2_W[i], gb_post[i])
    return (feat, ctrs)


# speedup vs baseline: 3.1263x; 1.0159x over previous
import functools

import jax
import jax.numpy as jnp
from jax import lax
from jax.experimental import pallas as pl
from jax.experimental.pallas import tpu as pltpu
from jax.experimental.pallas import tpu_sc as plsc

N = 50000
NMAP = 128
NSCALES = 6
E = 30000
EL = 10000
NLAYERS = 4
NSLOT = 14
EPS = 1e-5

BLK = 8448
NBLK = 6
NPAD = BLK * NBLK
TRASH = BLK
T = 128
ETOT = 2 * NSCALES * E + 2 * EL
EPADN = ((ETOT + T - 1) // T + NBLK) * T
ROWS_PER_SUB = BLK // 16
ZCH = 48
BR = 1000
GRID = N // BR


def _tcin_body(x_ref, w8_ref, b8_ref, wi2_ref, ws2_ref, gb_ref, wm1_ref,
               wm2_ref, gbm_ref, we_ref, o_ref, ze_ref):
    x = x_ref[...]
    a = jax.nn.relu(
        lax.dot_general(x, w8_ref[...], (((1,), (0,)), ((), ())),
                        preferred_element_type=jnp.float32) + b8_ref[...])
    h_in = lax.dot_general(a[:, :NMAP], wi2_ref[...], (((1,), (0,)), ((), ())),
                           preferred_element_type=jnp.float32)
    h_seg = lax.dot_general(a[:, NMAP:], ws2_ref[...], (((1,), (0,)), ((), ())),
                            preferred_element_type=jnp.float32)

    def gn(t, g, b):
        m = jnp.mean(t, axis=1, keepdims=True)
        v = jnp.mean((t - m) ** 2, axis=1, keepdims=True)
        return (t - m) * lax.rsqrt(v + EPS) * g + b

    gb = gb_ref[...]
    h_in = gn(h_in, gb[0:1, :], gb[1:2, :])
    h_seg = gn(h_seg, gb[2:3, :], gb[3:4, :])
    f = jax.nn.relu(h_in + h_seg)
    t = (lax.dot_general(f, wm1_ref[...], (((1,), (0,)), ((), ())),
                         preferred_element_type=jnp.float32)
         + lax.dot_general(x, wm2_ref[...], (((1,), (0,)), ((), ())),
                           preferred_element_type=jnp.float32))
    gbm = gbm_ref[...]
    f = jax.nn.relu(gn(t, gbm[0:1, :], gbm[1:2, :]))
    o_ref[...] = f
    for sl in range(NSLOT):
        ze_ref[sl] = lax.dot_general(f, we_ref[sl], (((1,), (0,)), ((), ())),
                                     preferred_element_type=jnp.float32)


def _tc_input(x, w8, b8, wi2, ws2, gb, wm1, wm2, gbm, we):
    return pl.pallas_call(
        _tcin_body,
        grid=(GRID,),
        in_specs=[
            pl.BlockSpec((BR, 8), lambda i: (i, 0)),
            pl.BlockSpec((8, 2 * NMAP), lambda i: (0, 0)),
            pl.BlockSpec((1, 2 * NMAP), lambda i: (0, 0)),
            pl.BlockSpec((NMAP, NMAP), lambda i: (0, 0)),
            pl.BlockSpec((NMAP, NMAP), lambda i: (0, 0)),
            pl.BlockSpec((4, NMAP), lambda i: (0, 0)),
            pl.BlockSpec((NMAP, NMAP), lambda i: (0, 0)),
            pl.BlockSpec((8, NMAP), lambda i: (0, 0)),
            pl.BlockSpec((2, NMAP), lambda i: (0, 0)),
            pl.BlockSpec((NSLOT, NMAP, NMAP), lambda i: (0, 0, 0)),
        ],
        out_specs=[
            pl.BlockSpec((BR, NMAP), lambda i: (i, 0)),
            pl.BlockSpec((NSLOT, BR, NMAP), lambda i: (0, i, 0)),
        ],
        out_shape=[
            jax.ShapeDtypeStruct((N, NMAP), jnp.float32),
            jax.ShapeDtypeStruct((NSLOT, N, NMAP), jnp.float32),
        ],
    )(x, w8, b8, wi2, ws2, gb, wm1, wm2, gbm, we)


def _tcmm_body(f_ref, we_ref, ze_ref):
    f = f_ref[...]
    for s in range(NSLOT):
        ze_ref[s] = lax.dot_general(f, we_ref[s], (((1,), (0,)), ((), ())),
                                    preferred_element_type=jnp.float32)


def _tc_matmul(feat, we):
    return pl.pallas_call(
        _tcmm_body,
        grid=(GRID,),
        in_specs=[
            pl.BlockSpec((BR, NMAP), lambda i: (i, 0)),
            pl.BlockSpec((NSLOT, NMAP, NMAP), lambda i: (0, 0, 0)),
        ],
        out_specs=pl.BlockSpec((NSLOT, BR, NMAP), lambda i: (0, i, 0)),
        out_shape=jax.ShapeDtypeStruct((NSLOT, N, NMAP), jnp.float32),
    )(feat, we)


def _sc_body(ze, sdgh, tloh, thih, out,
             sdg0, sdg1, sdg2, liv0, liv1, liv2, rows0, rows1, rows2,
             zbuf, tlov, thiv, spmem, sem0, sem1, sem2):
    c = lax.axis_index("c")
    w = lax.axis_index("s")
    pltpu.sync_copy(tloh, tlov)
    pltpu.sync_copy(thih, thiv)
    zero16 = jnp.zeros((16,), jnp.float32)
    for r in range(ZCH):
        for cc in range(8):
            zbuf[r, pl.ds(cc * 16, 16)] = zero16

    tlo_all = tlov[...]
    thi_all = thiv[...]
    sdgs = (sdg0, sdg1, sdg2)
    livs = (liv0, liv1, liv2)
    rowss = (rows0, rows1, rows2)
    sems = (sem0, sem1, sem2)

    def _fetch(t, slot):
        pltpu.sync_copy(sdgh.at[pl.ds(t * 2 * T, 2 * T)], sdgs[slot])
        pltpu.async_copy(ze.at[sdgs[slot].at[pl.ds(T, T)]], rowss[slot],
                         sems[slot])

    def _drain(base, slot):
        pltpu.make_async_copy(ze.at[sdgs[slot].at[pl.ds(T, T)]], rowss[slot],
                              sems[slot]).wait()
        for i in range(T // 16):
            d = sdgs[slot][pl.ds(i * 16, 16)]
            loc = d - base
            okm = (d >= base) & (d < base + BLK)
            livs[slot][pl.ds(i * 16, 16)] = jnp.where(okm, loc, TRASH)
        pltpu.sync_copy(rowss[slot], spmem.at[livs[slot]], add=True)

    for b in range(NBLK):
        @pl.when(c == b // (NBLK // 2))
        def _(b=b):
            base = b * BLK
            for kk in range(ROWS_PER_SUB // ZCH):
                pltpu.sync_copy(
                    zbuf, spmem.at[pl.ds(w * ROWS_PER_SUB + kk * ZCH, ZCH), :])
            plsc.subcore_barrier()

            tlo = tlo_all[b]
            thi = thi_all[b]
            nt = thi - tlo
            niter = jnp.maximum(nt - w + 15, 0) // 16

            @pl.when(niter > 0)
            def _():
                _fetch(tlo + w, 0)

                @pl.when(niter > 1)
                def _():
                    _fetch(tlo + w + 16, 1)

                def tri_body(k, carry):
                    j0 = 3 * k
                    for q, slot in ((0, 0), (1, 1), (2, 2)):
                        j = j0 + q

                        @pl.when(j < niter)
                        def _(j=j, slot=slot):
                            @pl.when(j + 2 < niter)
                            def _():
                                _fetch(tlo + w + (j + 2) * 16, (slot + 2) % 3)
                            _drain(base, slot)
                    return carry

                lax.fori_loop(0, (niter + 2) // 3, tri_body, 0)
            plsc.subcore_barrier()
            pltpu.sync_copy(
                spmem.at[pl.ds(w * ROWS_PER_SUB, ROWS_PER_SUB), :],
                out.at[pl.ds(base + w * ROWS_PER_SUB, ROWS_PER_SUB), :])
            plsc.subcore_barrier()


@functools.cache
def _sc_scatter_fn():
    return pl.kernel(
        _sc_body,
        out_type=jax.ShapeDtypeStruct((NPAD, NMAP), jnp.float32),
        mesh=plsc.VectorSubcoreMesh(core_axis_name="c", subcore_axis_name="s"),
        scratch_types=[
            pltpu.VMEM((2 * T,), jnp.int32),
            pltpu.VMEM((2 * T,), jnp.int32),
            pltpu.VMEM((2 * T,), jnp.int32),
            pltpu.VMEM((T,), jnp.int32),
            pltpu.VMEM((T,), jnp.int32),
            pltpu.VMEM((T,), jnp.int32),
            pltpu.VMEM((T, NMAP), jnp.float32),
            pltpu.VMEM((T, NMAP), jnp.float32),
            pltpu.VMEM((T, NMAP), jnp.float32),
            pltpu.VMEM((ZCH, NMAP), jnp.float32),
            pltpu.VMEM((16,), jnp.int32),
            pltpu.VMEM((16,), jnp.int32),
            pltpu.VMEM_SHARED((BLK + 8, NMAP), jnp.float32),
            pltpu.SemaphoreType.DMA,
            pltpu.SemaphoreType.DMA,
            pltpu.SemaphoreType.DMA,
        ],
    )


def _sc_scatter(ze, sdg, tlo, thi):
    return _sc_scatter_fn()(ze, sdg, tlo, thi)


def _tcpost_body(s_ref, res_ref, wc_ref, w2_ref, gb_ref, o_ref):
    res = res_ref[...]
    t = s_ref[...] + lax.dot_general(res, wc_ref[...], (((1,), (0,)), ((), ())),
                                     preferred_element_type=jnp.float32)

    def gn(t, g, b):
        m = jnp.mean(t, axis=1, keepdims=True)
        v = jnp.mean((t - m) ** 2, axis=1, keepdims=True)
        return (t - m) * lax.rsqrt(v + EPS) * g + b

    gb = gb_ref[...]
    a = jax.nn.relu(gn(t, gb[0:1, :], gb[1:2, :]))
    h = lax.dot_general(a, w2_ref[...], (((1,), (0,)), ((), ())),
                        preferred_element_type=jnp.float32)
    h = gn(h, gb[2:3, :], gb[3:4, :])
    o_ref[...] = jax.nn.relu(h + res)


def _tcpostmm_body(s_ref, res_ref, wc_ref, w2_ref, gb_ref, we_ref,
                   o_ref, ze_ref):
    res = res_ref[...]
    t = s_ref[...] + lax.dot_general(res, wc_ref[...], (((1,), (0,)), ((), ())),
                                     preferred_element_type=jnp.float32)

    def gn(t, g, b):
        m = jnp.mean(t, axis=1, keepdims=True)
        v = jnp.mean((t - m) ** 2, axis=1, keepdims=True)
        return (t - m) * lax.rsqrt(v + EPS) * g + b

    gb = gb_ref[...]
    a = jax.nn.relu(gn(t, gb[0:1, :], gb[1:2, :]))
    h = lax.dot_general(a, w2_ref[...], (((1,), (0,)), ((), ())),
                        preferred_element_type=jnp.float32)
    h = gn(h, gb[2:3, :], gb[3:4, :])
    f = jax.nn.relu(h + res)
    o_ref[...] = f
    for sl in range(NSLOT):
        ze_ref[sl] = lax.dot_general(f, we_ref[sl], (((1,), (0,)), ((), ())),
                                     preferred_element_type=jnp.float32)


def _tc_post_mm(s, res, wc, w2, gb, we):
    return pl.pallas_call(
        _tcpostmm_body,
        grid=(GRID,),
        in_specs=[
            pl.BlockSpec((BR, NMAP), lambda i: (i, 0)),
            pl.BlockSpec((BR, NMAP), lambda i: (i, 0)),
            pl.BlockSpec((NMAP, NMAP), lambda i: (0, 0)),
            pl.BlockSpec((NMAP, NMAP), lambda i: (0, 0)),
            pl.BlockSpec((4, NMAP), lambda i: (0, 0)),
            pl.BlockSpec((NSLOT, NMAP, NMAP), lambda i: (0, 0, 0)),
        ],
        out_specs=[
            pl.BlockSpec((BR, NMAP), lambda i: (i, 0)),
            pl.BlockSpec((NSLOT, BR, NMAP), lambda i: (0, i, 0)),
        ],
        out_shape=[
            jax.ShapeDtypeStruct((N, NMAP), jnp.float32),
            jax.ShapeDtypeStruct((NSLOT, N, NMAP), jnp.float32),
        ],
    )(s, res, wc, w2, gb, we)


def _tc_post(s, res, wc, w2, gb):
    return pl.pallas_call(
        _tcpost_body,
        grid=(GRID,),
        in_specs=[
            pl.BlockSpec((BR, NMAP), lambda i: (i, 0)),
            pl.BlockSpec((BR, NMAP), lambda i: (i, 0)),
            pl.BlockSpec((NMAP, NMAP), lambda i: (0, 0)),
            pl.BlockSpec((NMAP, NMAP), lambda i: (0, 0)),
            pl.BlockSpec((4, NMAP), lambda i: (0, 0)),
        ],
        out_specs=pl.BlockSpec((BR, NMAP), lambda i: (i, 0)),
        out_shape=jax.ShapeDtypeStruct((N, NMAP), jnp.float32),
    )(s, res, wc, w2, gb)


def kernel(control, pre, right, suc, turn, intersect, ctrs, feats, left,
           Wi1, bi1, Wi2, gi, bi, Ws1, bs1, Ws2, gs, bs, Wm, gm, bm,
           ctr_W, pre_W, suc_W, left_W, right_W, norm_g, norm_b,
           ctr2_W, ctr2_g, ctr2_b):
    f32 = jnp.float32
    x = jnp.concatenate([ctrs, feats, turn, control[:, None],
                         intersect[:, None]], axis=1).astype(f32)
    w8 = jnp.zeros((8, 2 * NMAP), f32)
    w8 = w8.at[0:2, :NMAP].set(Wi1).at[2:4, NMAP:].set(Ws1)
    b8 = jnp.concatenate([bi1, bs1])[None, :]
    gb_in = jnp.stack([gi, bi, gs, bs])
    wm1 = Wm[:NMAP]
    wm2 = jnp.zeros((8, NMAP), f32).at[4:8, :].set(Wm[NMAP:NMAP + 4])
    gbm = jnp.stack([gm, bm])
    we = jnp.concatenate([
        pre_W, suc_W, left_W[:, None], right_W[:, None]], axis=1)
    gb_post = jnp.stack([norm_g, norm_b, ctr2_g, ctr2_b], axis=1)

    dsts = jnp.concatenate([pre[:, 0].reshape(-1), suc[:, 0].reshape(-1),
                            left[0], right[0]])
    srcs = jnp.concatenate([pre[:, 1].reshape(-1), suc[:, 1].reshape(-1),
                            left[1], right[1]])
    slots = jnp.concatenate([
        jnp.repeat(jnp.arange(NSCALES, dtype=jnp.int32), E),
        NSCALES + jnp.repeat(jnp.arange(NSCALES, dtype=jnp.int32), E),
        jnp.full((EL,), 2 * NSCALES, jnp.int32),
        jnp.full((EL,), 2 * NSCALES + 1, jnp.int32)])
    g = slots * N + srcs
    sd, sg = lax.sort_key_val(dsts, g)
    sd = jnp.concatenate([sd, jnp.full((EPADN - ETOT,), NPAD - 1, jnp.int32)])
    sg = jnp.concatenate([sg, jnp.zeros((EPADN - ETOT,), jnp.int32)])
    sdg = jnp.stack([sd.reshape(-1, T), sg.reshape(-1, T)], axis=1).reshape(-1)
    bounds = jnp.searchsorted(
        sd, jnp.arange(NBLK + 1, dtype=jnp.int32) * BLK).astype(jnp.int32)
    tlo = jnp.zeros((16,), jnp.int32).at[:NBLK].set(bounds[:NBLK] // T)
    thi = jnp.zeros((16,), jnp.int32).at[:NBLK].set(
        (bounds[1:] + T - 1) // T)

    feat, ze = _tc_input(x, w8, b8, Wi2, Ws2, gb_in, wm1, wm2, gbm, we[0])
    for i in range(NLAYERS):
        sacc = _sc_scatter(ze.reshape(NSLOT * N, NMAP), sdg, tlo, thi)
        if i < NLAYERS - 1:
            feat, ze = _tc_post_mm(sacc, feat, ctr_W[i], ctr2_W[i],
                                   gb_post[i], we[i + 1])
        else:
            feat = _tc_post(sacc, feat, ctr_W[i], ctr2_W[i], gb_post[i])
    return (feat, ctrs)


# speedup vs baseline: 3.1279x; 1.0005x over previous
import functools

import jax
import jax.numpy as jnp
from jax import lax
from jax.experimental import pallas as pl
from jax.experimental.pallas import tpu as pltpu
from jax.experimental.pallas import tpu_sc as plsc

N = 50000
NMAP = 128
NSCALES = 6
E = 30000
EL = 10000
NLAYERS = 4
NSLOT = 14
EPS = 1e-5

BLK = 8448
NBLK = 6
NPAD = BLK * NBLK
TRASH = BLK
T = 128
ETOT = 2 * NSCALES * E + 2 * EL
EPADN = ((ETOT + T - 1) // T + NBLK) * T
ROWS_PER_SUB = BLK // 16
ZCH = 48
BR = 1000
GRID = N // BR


def _tcin_body(x_ref, w8_ref, b8_ref, wi2_ref, ws2_ref, gb_ref, wm1_ref,
               wm2_ref, gbm_ref, we_ref, o_ref, ze_ref):
    x = x_ref[...]
    a = jax.nn.relu(
        lax.dot_general(x, w8_ref[...], (((1,), (0,)), ((), ())),
                        preferred_element_type=jnp.float32) + b8_ref[...])
    h_in = lax.dot_general(a[:, :NMAP], wi2_ref[...], (((1,), (0,)), ((), ())),
                           preferred_element_type=jnp.float32)
    h_seg = lax.dot_general(a[:, NMAP:], ws2_ref[...], (((1,), (0,)), ((), ())),
                            preferred_element_type=jnp.float32)

    def gn(t, g, b):
        m = jnp.mean(t, axis=1, keepdims=True)
        v = jnp.mean((t - m) ** 2, axis=1, keepdims=True)
        return (t - m) * lax.rsqrt(v + EPS) * g + b

    gb = gb_ref[...]
    h_in = gn(h_in, gb[0:1, :], gb[1:2, :])
    h_seg = gn(h_seg, gb[2:3, :], gb[3:4, :])
    f = jax.nn.relu(h_in + h_seg)
    t = (lax.dot_general(f, wm1_ref[...], (((1,), (0,)), ((), ())),
                         preferred_element_type=jnp.float32)
         + lax.dot_general(x, wm2_ref[...], (((1,), (0,)), ((), ())),
                           preferred_element_type=jnp.float32))
    gbm = gbm_ref[...]
    f = jax.nn.relu(gn(t, gbm[0:1, :], gbm[1:2, :]))
    o_ref[...] = f
    for sl in range(NSLOT):
        ze_ref[sl] = lax.dot_general(f, we_ref[sl], (((1,), (0,)), ((), ())),
                                     preferred_element_type=jnp.float32)


def _tc_input(x, w8, b8, wi2, ws2, gb, wm1, wm2, gbm, we):
    return pl.pallas_call(
        _tcin_body,
        grid=(GRID,),
        in_specs=[
            pl.BlockSpec((BR, 8), lambda i: (i, 0)),
            pl.BlockSpec((8, 2 * NMAP), lambda i: (0, 0)),
            pl.BlockSpec((1, 2 * NMAP), lambda i: (0, 0)),
            pl.BlockSpec((NMAP, NMAP), lambda i: (0, 0)),
            pl.BlockSpec((NMAP, NMAP), lambda i: (0, 0)),
            pl.BlockSpec((4, NMAP), lambda i: (0, 0)),
            pl.BlockSpec((NMAP, NMAP), lambda i: (0, 0)),
            pl.BlockSpec((8, NMAP), lambda i: (0, 0)),
            pl.BlockSpec((2, NMAP), lambda i: (0, 0)),
            pl.BlockSpec((NSLOT, NMAP, NMAP), lambda i: (0, 0, 0)),
        ],
        out_specs=[
            pl.BlockSpec((BR, NMAP), lambda i: (i, 0)),
            pl.BlockSpec((NSLOT, BR, NMAP), lambda i: (0, i, 0)),
        ],
        out_shape=[
            jax.ShapeDtypeStruct((N, NMAP), jnp.float32),
            jax.ShapeDtypeStruct((NSLOT, N, NMAP), jnp.float32),
        ],
    )(x, w8, b8, wi2, ws2, gb, wm1, wm2, gbm, we)


def _sc_body(ze, sdgh, tloh, thih, out,
             sdg0, sdg1, sdg2, liv0, liv1, liv2, rows0, rows1, rows2,
             zbuf, tlov, thiv, spmem, sem0, sem1, sem2):
    c = lax.axis_index("c")
    w = lax.axis_index("s")
    pltpu.sync_copy(tloh, tlov)
    pltpu.sync_copy(thih, thiv)
    zero16 = jnp.zeros((16,), jnp.float32)
    for r in range(ZCH):
        for cc in range(8):
            zbuf[r, pl.ds(cc * 16, 16)] = zero16

    tlo_all = tlov[...]
    thi_all = thiv[...]
    sdgs = (sdg0, sdg1, sdg2)
    livs = (liv0, liv1, liv2)
    rowss = (rows0, rows1, rows2)
    sems = (sem0, sem1, sem2)

    def _fetch(t, slot):
        pltpu.sync_copy(sdgh.at[pl.ds(t * 2 * T, 2 * T)], sdgs[slot])
        pltpu.async_copy(ze.at[sdgs[slot].at[pl.ds(T, T)]], rowss[slot],
                         sems[slot])

    def _drain(base, slot):
        pltpu.make_async_copy(ze.at[sdgs[slot].at[pl.ds(T, T)]], rowss[slot],
                              sems[slot]).wait()
        for i in range(T // 16):
            d = sdgs[slot][pl.ds(i * 16, 16)]
            loc = d - base
            okm = (d >= base) & (d < base + BLK)
            livs[slot][pl.ds(i * 16, 16)] = jnp.where(okm, loc, TRASH)
        pltpu.sync_copy(rowss[slot], spmem.at[livs[slot]], add=True)

    for b in range(NBLK):
        @pl.when(c == b // (NBLK // 2))
        def _(b=b):
            base = b * BLK
            for kk in range(ROWS_PER_SUB // ZCH):
                pltpu.sync_copy(
                    zbuf, spmem.at[pl.ds(w * ROWS_PER_SUB + kk * ZCH, ZCH), :])
            plsc.subcore_barrier()

            tlo = tlo_all[b]
            thi = thi_all[b]
            nt = thi - tlo
            niter = jnp.maximum(nt - w + 15, 0) // 16

            @pl.when(niter > 0)
            def _():
                _fetch(tlo + w, 0)

                @pl.when(niter > 1)
                def _():
                    _fetch(tlo + w + 16, 1)

                def tri_body(k, carry):
                    j0 = 3 * k
                    for q, slot in ((0, 0), (1, 1), (2, 2)):
                        j = j0 + q

                        @pl.when(j < niter)
                        def _(j=j, slot=slot):
                            @pl.when(j + 2 < niter)
                            def _():
                                _fetch(tlo + w + (j + 2) * 16, (slot + 2) % 3)
                            _drain(base, slot)
                    return carry

                lax.fori_loop(0, (niter + 2) // 3, tri_body, 0)
            plsc.subcore_barrier()
            pltpu.sync_copy(
                spmem.at[pl.ds(w * ROWS_PER_SUB, ROWS_PER_SUB), :],
                out.at[pl.ds(base + w * ROWS_PER_SUB, ROWS_PER_SUB), :])
            plsc.subcore_barrier()


@functools.cache
def _sc_scatter_fn():
    return pl.kernel(
        _sc_body,
        out_type=jax.ShapeDtypeStruct((NPAD, NMAP), jnp.float32),
        mesh=plsc.VectorSubcoreMesh(core_axis_name="c", subcore_axis_name="s"),
        scratch_types=[
            pltpu.VMEM((2 * T,), jnp.int32),
            pltpu.VMEM((2 * T,), jnp.int32),
            pltpu.VMEM((2 * T,), jnp.int32),
            pltpu.VMEM((T,), jnp.int32),
            pltpu.VMEM((T,), jnp.int32),
            pltpu.VMEM((T,), jnp.int32),
            pltpu.VMEM((T, NMAP), jnp.float32),
            pltpu.VMEM((T, NMAP), jnp.float32),
            pltpu.VMEM((T, NMAP), jnp.float32),
            pltpu.VMEM((ZCH, NMAP), jnp.float32),
            pltpu.VMEM((16,), jnp.int32),
            pltpu.VMEM((16,), jnp.int32),
            pltpu.VMEM_SHARED((BLK + 8, NMAP), jnp.float32),
            pltpu.SemaphoreType.DMA,
            pltpu.SemaphoreType.DMA,
            pltpu.SemaphoreType.DMA,
        ],
    )


def _sc_scatter(ze, sdg, tlo, thi):
    return _sc_scatter_fn()(ze, sdg, tlo, thi)


def _tcpost_body(s_ref, res_ref, wc_ref, w2_ref, gb_ref, o_ref):
    res = res_ref[...]
    t = s_ref[...] + lax.dot_general(res, wc_ref[...], (((1,), (0,)), ((), ())),
                                     preferred_element_type=jnp.float32)

    def gn(t, g, b):
        m = jnp.mean(t, axis=1, keepdims=True)
        v = jnp.mean((t - m) ** 2, axis=1, keepdims=True)
        return (t - m) * lax.rsqrt(v + EPS) * g + b

    gb = gb_ref[...]
    a = jax.nn.relu(gn(t, gb[0:1, :], gb[1:2, :]))
    h = lax.dot_general(a, w2_ref[...], (((1,), (0,)), ((), ())),
                        preferred_element_type=jnp.float32)
    h = gn(h, gb[2:3, :], gb[3:4, :])
    o_ref[...] = jax.nn.relu(h + res)


def _tcpostmm_body(s_ref, res_ref, wc_ref, w2_ref, gb_ref, we_ref,
                   o_ref, ze_ref):
    res = res_ref[...]
    t = s_ref[...] + lax.dot_general(res, wc_ref[...], (((1,), (0,)), ((), ())),
                                     preferred_element_type=jnp.float32)

    def gn(t, g, b):
        m = jnp.mean(t, axis=1, keepdims=True)
        v = jnp.mean((t - m) ** 2, axis=1, keepdims=True)
        return (t - m) * lax.rsqrt(v + EPS) * g + b

    gb = gb_ref[...]
    a = jax.nn.relu(gn(t, gb[0:1, :], gb[1:2, :]))
    h = lax.dot_general(a, w2_ref[...], (((1,), (0,)), ((), ())),
                        preferred_element_type=jnp.float32)
    h = gn(h, gb[2:3, :], gb[3:4, :])
    f = jax.nn.relu(h + res)
    o_ref[...] = f
    for sl in range(NSLOT):
        ze_ref[sl] = lax.dot_general(f, we_ref[sl], (((1,), (0,)), ((), ())),
                                     preferred_element_type=jnp.float32)


def _tc_post_mm(s, res, wc, w2, gb, we):
    return pl.pallas_call(
        _tcpostmm_body,
        grid=(GRID,),
        in_specs=[
            pl.BlockSpec((BR, NMAP), lambda i: (i, 0)),
            pl.BlockSpec((BR, NMAP), lambda i: (i, 0)),
            pl.BlockSpec((NMAP, NMAP), lambda i: (0, 0)),
            pl.BlockSpec((NMAP, NMAP), lambda i: (0, 0)),
            pl.BlockSpec((4, NMAP), lambda i: (0, 0)),
            pl.BlockSpec((NSLOT, NMAP, NMAP), lambda i: (0, 0, 0)),
        ],
        out_specs=[
            pl.BlockSpec((BR, NMAP), lambda i: (i, 0)),
            pl.BlockSpec((NSLOT, BR, NMAP), lambda i: (0, i, 0)),
        ],
        out_shape=[
            jax.ShapeDtypeStruct((N, NMAP), jnp.float32),
            jax.ShapeDtypeStruct((NSLOT, N, NMAP), jnp.float32),
        ],
    )(s, res, wc, w2, gb, we)


def _tc_post(s, res, wc, w2, gb):
    return pl.pallas_call(
        _tcpost_body,
        grid=(GRID,),
        in_specs=[
            pl.BlockSpec((BR, NMAP), lambda i: (i, 0)),
            pl.BlockSpec((BR, NMAP), lambda i: (i, 0)),
            pl.BlockSpec((NMAP, NMAP), lambda i: (0, 0)),
            pl.BlockSpec((NMAP, NMAP), lambda i: (0, 0)),
            pl.BlockSpec((4, NMAP), lambda i: (0, 0)),
        ],
        out_specs=pl.BlockSpec((BR, NMAP), lambda i: (i, 0)),
        out_shape=jax.ShapeDtypeStruct((N, NMAP), jnp.float32),
    )(s, res, wc, w2, gb)


def kernel(control, pre, right, suc, turn, intersect, ctrs, feats, left,
           Wi1, bi1, Wi2, gi, bi, Ws1, bs1, Ws2, gs, bs, Wm, gm, bm,
           ctr_W, pre_W, suc_W, left_W, right_W, norm_g, norm_b,
           ctr2_W, ctr2_g, ctr2_b):
    f32 = jnp.float32
    x = jnp.concatenate([ctrs, feats, turn, control[:, None],
                         intersect[:, None]], axis=1).astype(f32)
    w8 = jnp.zeros((8, 2 * NMAP), f32)
    w8 = w8.at[0:2, :NMAP].set(Wi1).at[2:4, NMAP:].set(Ws1)
    b8 = jnp.concatenate([bi1, bs1])[None, :]
    gb_in = jnp.stack([gi, bi, gs, bs])
    wm1 = Wm[:NMAP]
    wm2 = jnp.zeros((8, NMAP), f32).at[4:8, :].set(Wm[NMAP:NMAP + 4])
    gbm = jnp.stack([gm, bm])
    we = jnp.concatenate([
        pre_W, suc_W, left_W[:, None], right_W[:, None]], axis=1)
    gb_post = jnp.stack([norm_g, norm_b, ctr2_g, ctr2_b], axis=1)

    dsts = jnp.concatenate([pre[:, 0].reshape(-1), suc[:, 0].reshape(-1),
                            left[0], right[0]])
    srcs = jnp.concatenate([pre[:, 1].reshape(-1), suc[:, 1].reshape(-1),
                            left[1], right[1]])
    slots = jnp.concatenate([
        jnp.repeat(jnp.arange(NSCALES, dtype=jnp.int32), E),
        NSCALES + jnp.repeat(jnp.arange(NSCALES, dtype=jnp.int32), E),
        jnp.full((EL,), 2 * NSCALES, jnp.int32),
        jnp.full((EL,), 2 * NSCALES + 1, jnp.int32)])
    g = slots * N + srcs
    sd, sg = lax.sort_key_val(dsts, g)
    sd = jnp.concatenate([sd, jnp.full((EPADN - ETOT,), NPAD - 1, jnp.int32)])
    sg = jnp.concatenate([sg, jnp.zeros((EPADN - ETOT,), jnp.int32)])
    sdg = jnp.stack([sd.reshape(-1, T), sg.reshape(-1, T)], axis=1).reshape(-1)
    bounds = jnp.searchsorted(
        sd, jnp.arange(NBLK + 1, dtype=jnp.int32) * BLK).astype(jnp.int32)
    tlo = jnp.zeros((16,), jnp.int32).at[:NBLK].set(bounds[:NBLK] // T)
    thi = jnp.zeros((16,), jnp.int32).at[:NBLK].set(
        (bounds[1:] + T - 1) // T)

    feat, ze = _tc_input(x, w8, b8, Wi2, Ws2, gb_in, wm1, wm2, gbm, we[0])
    for i in range(NLAYERS):
        sacc = _sc_scatter(ze.reshape(NSLOT * N, NMAP), sdg, tlo, thi)
        if i < NLAYERS - 1:
            feat, ze = _tc_post_mm(sacc, feat, ctr_W[i], ctr2_W[i],
                                   gb_post[i], we[i + 1])
        else:
            feat = _tc_post(sacc, feat, ctr_W[i], ctr2_W[i], gb_post[i])
    return (feat, ctrs)
